# Initial kernel scaffold; baseline (speedup 1.0000x reference)
import jax
import jax.numpy as jnp
from jax.experimental import pallas as pl


def kernel(x, edge_index, edge_attr, batch, ln_node_g, ln_node_b, ln_edge_g, ln_edge_b, W_msg0, b_msg0, W_upd0, b_upd0, W_msg1, b_msg1, W_upd1, b_upd1, W_msg2, b_msg2, W_upd2, b_upd2, W_lin1, b_lin1, W_fin, b_fin):
    raise NotImplementedError("write your pallas kernel here")


# speedup vs baseline: 2.1119x; 2.1119x over previous
import functools

import jax
import jax.numpy as jnp
from jax import lax
from jax.experimental import pallas as pl
from jax.experimental.pallas import tpu as pltpu
from jax.experimental.pallas import tpu_sc as plsc

N = 10000
NP = 10240
E = 320000
EP = 323584
D = 128
DE = 16
H = 128
G = 64
C = 10
SLOPE = 0.2

KE = 128
CHUNKS_PER_WORKER = 79
NW = 32
ROWS_PER_SUB = NP // 16
TB = 512
TEB = 2048


def _ln_z_body(x_ref, g_ref, b_ref, wmx_ref, x0_ref, z0_ref):
    x = x_ref[...]
    mu = jnp.mean(x, axis=1, keepdims=True)
    var = jnp.mean((x - mu) ** 2, axis=1, keepdims=True)
    xn = (x - mu) * jax.lax.rsqrt(var + 1e-5) * g_ref[...] + b_ref[...]
    x0_ref[...] = xn
    z0_ref[...] = jnp.dot(xn, wmx_ref[...], preferred_element_type=jnp.float32)


def _ln_z(x_p, g, b, wmx):
    grid = NP // TB
    return pl.pallas_call(
        _ln_z_body,
        grid=(grid,),
        in_specs=[
            pl.BlockSpec((TB, D), lambda i: (i, 0)),
            pl.BlockSpec((1, D), lambda i: (0, 0)),
            pl.BlockSpec((1, D), lambda i: (0, 0)),
            pl.BlockSpec((D, H), lambda i: (0, 0)),
        ],
        out_specs=[
            pl.BlockSpec((TB, D), lambda i: (i, 0)),
            pl.BlockSpec((TB, H), lambda i: (i, 0)),
        ],
        out_shape=[
            jax.ShapeDtypeStruct((NP, D), jnp.float32),
            jax.ShapeDtypeStruct((NP, H), jnp.float32),
        ],
    )(x_p, g, b, wmx)


def _eterm_body(ea_ref, g_ref, b_ref, w0_ref, b0_ref, w1_ref, b1_ref,
                w2_ref, b2_ref, e0_ref, e1_ref, e2_ref):
    ea = ea_ref[...]
    mu = jnp.mean(ea, axis=1, keepdims=True)
    var = jnp.mean((ea - mu) ** 2, axis=1, keepdims=True)
    ean = (ea - mu) * jax.lax.rsqrt(var + 1e-5) * g_ref[...] + b_ref[...]
    e0_ref[...] = jnp.dot(ean, w0_ref[...], preferred_element_type=jnp.float32) + b0_ref[...]
    e1_ref[...] = jnp.dot(ean, w1_ref[...], preferred_element_type=jnp.float32) + b1_ref[...]
    e2_ref[...] = jnp.dot(ean, w2_ref[...], preferred_element_type=jnp.float32) + b2_ref[...]


def _eterms(ea_p, g, b, we0, bm0, we1, bm1, we2, bm2):
    grid = EP // TEB
    wspec = pl.BlockSpec((DE, H), lambda i: (0, 0))
    bspec = pl.BlockSpec((1, H), lambda i: (0, 0))
    ospec = pl.BlockSpec((TEB, H), lambda i: (i, 0))
    return pl.pallas_call(
        _eterm_body,
        grid=(grid,),
        in_specs=[
            pl.BlockSpec((TEB, DE), lambda i: (i, 0)),
            pl.BlockSpec((1, DE), lambda i: (0, 0)),
            pl.BlockSpec((1, DE), lambda i: (0, 0)),
            wspec, bspec, wspec, bspec, wspec, bspec,
        ],
        out_specs=[ospec, ospec, ospec],
        out_shape=[jax.ShapeDtypeStruct((EP, H), jnp.float32)] * 3,
    )(ea_p, g, b, we0, bm0, we1, bm1, we2, bm2)


def _sc_msg_body(z_hbm, et_hbm, src_hbm, dst_hbm, out_hbm,
                 src_v, dst_v, zbuf, ebuf, agg_sh, sem):
    cid = lax.axis_index("c")
    sid = lax.axis_index("s")
    w = cid * 16 + sid

    zv = jnp.zeros((16,), jnp.float32)

    def zrow(r, _):
        for j in range(H // 16):
            zbuf[r, pl.ds(j * 16, 16)] = zv
        return 0

    lax.fori_loop(0, KE, zrow, 0)
    for k in range(ROWS_PER_SUB // KE):
        pltpu.sync_copy(zbuf, agg_sh.at[pl.ds(sid * ROWS_PER_SUB + k * KE, KE)])
    plsc.subcore_barrier()

    def chunk(i, _):
        base = (w * CHUNKS_PER_WORKER + i) * KE
        pltpu.sync_copy(src_hbm.at[pl.ds(base, KE)], src_v)
        pltpu.sync_copy(dst_hbm.at[pl.ds(base, KE)], dst_v)
        pltpu.sync_copy(et_hbm.at[pl.ds(base, KE)], ebuf)
        pltpu.async_copy(z_hbm.at[src_v], zbuf, sem).wait()

        def row(r, _):
            for j in range(H // 16):
                sl = pl.ds(j * 16, 16)
                ebuf[r, sl] = jnp.maximum(zbuf[r, sl] + ebuf[r, sl], 0.0)
            return 0

        lax.fori_loop(0, KE, row, 0)
        pltpu.sync_copy(ebuf, agg_sh.at[dst_v], add=True)
        return 0

    lax.fori_loop(0, CHUNKS_PER_WORKER, chunk, 0)
    plsc.subcore_barrier()
    pltpu.sync_copy(agg_sh.at[pl.ds(sid * ROWS_PER_SUB, ROWS_PER_SUB)],
                    out_hbm.at[cid, pl.ds(sid * ROWS_PER_SUB, ROWS_PER_SUB)])


def _sc_msg(z, et, src, dst):
    k = pl.kernel(
        _sc_msg_body,
        out_type=jax.ShapeDtypeStruct((2, NP, H), jnp.float32),
        mesh=plsc.VectorSubcoreMesh(core_axis_name="c", subcore_axis_name="s"),
        scratch_types=[
            pltpu.VMEM((KE,), jnp.int32),
            pltpu.VMEM((KE,), jnp.int32),
            pltpu.VMEM((KE, H), jnp.float32),
            pltpu.VMEM((KE, H), jnp.float32),
            pltpu.VMEM_SHARED((NP, H), jnp.float32),
            pltpu.SemaphoreType.DMA,
        ],
    )
    return k(z, et, src, dst)


def _upd_body(x_ref, a0_ref, a1_ref, bat_ref, wux_ref, wua_ref, bu_ref,
              wmx_ref, xn_ref, zn_ref, pool_ref):
    x = x_ref[...]
    a = a0_ref[...] + a1_ref[...]
    xn = jnp.dot(x, wux_ref[...], preferred_element_type=jnp.float32)
    xn = xn + jnp.dot(a, wua_ref[...], preferred_element_type=jnp.float32)
    xn = jnp.maximum(xn + bu_ref[...], 0.0)
    xn_ref[...] = xn
    zn_ref[...] = jnp.dot(xn, wmx_ref[...], preferred_element_type=jnp.float32)
    ids = bat_ref[...]
    oh = (ids == lax.broadcasted_iota(jnp.int32, (TB, G), 1)).astype(jnp.float32)
    contrib = lax.dot_general(oh, xn, (((0,), (0,)), ((), ())),
                              preferred_element_type=jnp.float32)

    @pl.when(pl.program_id(0) == 0)
    def _():
        pool_ref[...] = jnp.zeros_like(pool_ref)

    pool_ref[...] += contrib


def _upd_pool_body(x_ref, a0_ref, a1_ref, bat_ref, wux_ref, wua_ref, bu_ref,
                   pool_ref):
    x = x_ref[...]
    a = a0_ref[...] + a1_ref[...]
    xn = jnp.dot(x, wux_ref[...], preferred_element_type=jnp.float32)
    xn = xn + jnp.dot(a, wua_ref[...], preferred_element_type=jnp.float32)
    xn = jnp.maximum(xn + bu_ref[...], 0.0)
    ids = bat_ref[...]
    oh = (ids == lax.broadcasted_iota(jnp.int32, (TB, G), 1)).astype(jnp.float32)
    contrib = lax.dot_general(oh, xn, (((0,), (0,)), ((), ())),
                              preferred_element_type=jnp.float32)

    @pl.when(pl.program_id(0) == 0)
    def _():
        pool_ref[...] = jnp.zeros_like(pool_ref)

    pool_ref[...] += contrib


def _update(x_p, agg0, agg1, bat_p, wux, wua, bu, wmx_next):
    grid = NP // TB
    nspec = pl.BlockSpec((TB, H), lambda i: (i, 0))
    wspec = pl.BlockSpec((H, H), lambda i: (0, 0))
    pspec = pl.BlockSpec((G, H), lambda i: (0, 0))
    in_specs = [
        nspec, nspec, nspec,
        pl.BlockSpec((TB, 1), lambda i: (i, 0)),
        wspec, wspec,
        pl.BlockSpec((1, H), lambda i: (0, 0)),
    ]
    if wmx_next is not None:
        return pl.pallas_call(
            _upd_body,
            grid=(grid,),
            in_specs=in_specs + [wspec],
            out_specs=[nspec, nspec, pspec],
            out_shape=[
                jax.ShapeDtypeStruct((NP, H), jnp.float32),
                jax.ShapeDtypeStruct((NP, H), jnp.float32),
                jax.ShapeDtypeStruct((G, H), jnp.float32),
            ],
        )(x_p, agg0, agg1, bat_p, wux, wua, bu, wmx_next)
    return pl.pallas_call(
        _upd_pool_body,
        grid=(grid,),
        in_specs=in_specs,
        out_specs=pspec,
        out_shape=jax.ShapeDtypeStruct((G, H), jnp.float32),
    )(x_p, agg0, agg1, bat_p, wux, wua, bu)


def _head_body(p0_ref, p1_ref, p2_ref, bat_ref, w1_ref, b1_ref,
               wf_ref, bf_ref, out_ref):
    ids = bat_ref[...]
    oh = (ids == lax.broadcasted_iota(jnp.int32, (NP, G), 1)).astype(jnp.float32)
    counts = jnp.maximum(jnp.sum(oh, axis=0), 1.0)
    inv = (1.0 / counts)[:, None]
    h = jnp.concatenate([p0_ref[...] * inv, p1_ref[...] * inv, p2_ref[...] * inv],
                        axis=1)
    y = jnp.dot(h, w1_ref[...], preferred_element_type=jnp.float32) + b1_ref[...]
    y = jnp.where(y >= 0.0, y, SLOPE * y)
    out_ref[...] = jnp.dot(y, wf_ref[...], preferred_element_type=jnp.float32) + bf_ref[...]


def _head(p0, p1, p2, bat_p, w1, b1, wf_p, bf_p):
    return pl.pallas_call(
        _head_body,
        out_shape=jax.ShapeDtypeStruct((G, 128), jnp.float32),
    )(p0, p1, p2, bat_p, w1, b1, wf_p, bf_p)


def kernel(x, edge_index, edge_attr, batch,
           ln_node_g, ln_node_b, ln_edge_g, ln_edge_b,
           W_msg0, b_msg0, W_upd0, b_upd0,
           W_msg1, b_msg1, W_upd1, b_upd1,
           W_msg2, b_msg2, W_upd2, b_upd2,
           W_lin1, b_lin1, W_fin, b_fin):
    f32 = jnp.float32
    x_p = jnp.pad(x, ((0, NP - N), (0, 0)))
    ea_p = jnp.pad(edge_attr, ((0, EP - E), (0, 0)))
    src_p = jnp.pad(edge_index[0], (0, EP - E), constant_values=NP - 1)
    dst_p = jnp.pad(edge_index[1], (0, EP - E), constant_values=NP - 1)
    bat_p = jnp.pad(batch, (0, NP - N), constant_values=G)[:, None]

    g_n = ln_node_g[None, :].astype(f32)
    b_n = ln_node_b[None, :].astype(f32)
    g_e = ln_edge_g[None, :].astype(f32)
    b_e = ln_edge_b[None, :].astype(f32)

    wmx = [W_msg0[:D], W_msg1[:D], W_msg2[:D]]
    wme = [W_msg0[D:], W_msg1[D:], W_msg2[D:]]
    bm = [b_msg0[None, :], b_msg1[None, :], b_msg2[None, :]]
    wux = [W_upd0[:D], W_upd1[:D], W_upd2[:D]]
    wua = [W_upd0[D:], W_upd1[D:], W_upd2[D:]]
    bu = [b_upd0[None, :], b_upd1[None, :], b_upd2[None, :]]

    x0, z = _ln_z(x_p, g_n, b_n, wmx[0])
    et0, et1, et2 = _eterms(ea_p, g_e, b_e, wme[0], bm[0], wme[1], bm[1],
                            wme[2], bm[2])
    ets = [et0, et1, et2]

    pooled = []
    x_cur = x0
    for l in range(3):
        aggp = _sc_msg(z, ets[l], src_p, dst_p)
        if l < 2:
            x_cur, z, pool = _update(x_cur, aggp[0], aggp[1], bat_p,
                                     wux[l], wua[l], bu[l], wmx[l + 1])
        else:
            pool = _update(x_cur, aggp[0], aggp[1], bat_p,
                           wux[l], wua[l], bu[l], None)
        pooled.append(pool)

    wf_p = jnp.pad(W_fin, ((0, 0), (0, 128 - C)))
    bf_p = jnp.pad(b_fin, (0, 128 - C))[None, :]
    out = _head(pooled[0], pooled[1], pooled[2], bat_p,
                W_lin1, b_lin1[None, :], wf_p, bf_p)
    return out[:, :C]


# speedup vs baseline: 3.7117x; 1.7576x over previous
import jax
import jax.numpy as jnp
from jax import lax
from jax.experimental import pallas as pl
from jax.experimental.pallas import tpu as pltpu
from jax.experimental.pallas import tpu_sc as plsc

N = 10000
NP = 10240
E = 320000
EP = 327680
D = 128
DE = 16
H = 128
HH = H // 2
G = 64
C = 10
SLOPE = 0.2

KE = 128
CPW = 80
NREAL = E // KE
NPA = 10112
ROWS_PER_SUB = NPA // 16
TB = 512
TEB = 2048


def _ln_z_body(x_ref, g_ref, b_ref, wmx_ref, x0_ref, z_ref):
    x = x_ref[...]
    mu = jnp.mean(x, axis=1, keepdims=True)
    var = jnp.mean((x - mu) ** 2, axis=1, keepdims=True)
    xn = (x - mu) * jax.lax.rsqrt(var + 1e-5) * g_ref[...] + b_ref[...]
    x0_ref[...] = xn
    z_ref[...] = jnp.dot(xn, wmx_ref[...], preferred_element_type=jnp.float32)


def _ln_z(x_p, g, b, wmx):
    grid = NP // TB
    return pl.pallas_call(
        _ln_z_body,
        grid=(grid,),
        in_specs=[
            pl.BlockSpec((TB, D), lambda i: (i, 0)),
            pl.BlockSpec((1, D), lambda i: (0, 0)),
            pl.BlockSpec((1, D), lambda i: (0, 0)),
            pl.BlockSpec((D, H), lambda i: (0, 0)),
        ],
        out_specs=[
            pl.BlockSpec((TB, D), lambda i: (i, 0)),
            pl.BlockSpec((TB, H), lambda i: (i, 0)),
        ],
        out_shape=[
            jax.ShapeDtypeStruct((NP, D), jnp.float32),
            jax.ShapeDtypeStruct((NP, H), jnp.float32),
        ],
    )(x_p, g, b, wmx)


def _eterm_body(ea_ref, g_ref, b_ref, w_ref, bm_ref, e2_ref):
    ea = ea_ref[...]
    mu = jnp.mean(ea, axis=1, keepdims=True)
    var = jnp.mean((ea - mu) ** 2, axis=1, keepdims=True)
    ean = (ea - mu) * jax.lax.rsqrt(var + 1e-5) * g_ref[...] + b_ref[...]
    e2_ref[...] = jnp.dot(ean, w_ref[...], preferred_element_type=jnp.float32) + bm_ref[...]


def _eterm(ea_p, g, b, we, bm):
    grid = EP // TEB
    return pl.pallas_call(
        _eterm_body,
        grid=(grid,),
        in_specs=[
            pl.BlockSpec((TEB, DE), lambda i: (i, 0)),
            pl.BlockSpec((1, DE), lambda i: (0, 0)),
            pl.BlockSpec((1, DE), lambda i: (0, 0)),
            pl.BlockSpec((DE, H), lambda i: (0, 0)),
            pl.BlockSpec((1, H), lambda i: (0, 0)),
        ],
        out_specs=pl.BlockSpec((TEB, H), lambda i: (i, 0)),
        out_shape=jax.ShapeDtypeStruct((EP, H), jnp.float32),
    )(ea_p, g, b, we, bm)


def _sc_msg_body(z_hbm, et_hbm, sd_hbm, out_hbm,
                 sdv, etb, zb, agg_sh, sem_g, sem_et):
    cid = lax.axis_index("c")
    sid = lax.axis_index("s")
    w = cid * 16 + sid
    c0 = w * CPW
    zv = jnp.zeros((16,), jnp.float32)

    def zrow(r, _):
        for j in range(H // 16):
            etb[r, pl.ds(j * 16, 16)] = zv
        return 0

    lax.fori_loop(0, KE, zrow, 0)
    r0 = sid * ROWS_PER_SUB
    for k in range(4):
        pltpu.sync_copy(etb, agg_sh.at[pl.ds(r0 + k * KE, KE)])
    pltpu.sync_copy(etb.at[pl.ds(0, ROWS_PER_SUB - 4 * KE)],
                    agg_sh.at[pl.ds(r0 + 4 * KE, ROWS_PER_SUB - 4 * KE)])
    plsc.subcore_barrier()

    def et_copy(c):
        return pltpu.make_async_copy(
            et_hbm.at[pl.ds((c0 + c) * KE, KE)], etb, sem_et)

    def gather(c, s):
        return pltpu.make_async_copy(z_hbm.at[sdv[s].at[0]], zb[s], sem_g[s])

    pltpu.sync_copy(sd_hbm.at[c0], sdv[0])
    gather(0, 0).start()
    et_copy(0).start()

    def phase(c, s):
        gc = c0 + c
        s1 = 1 - s

        @pl.when(c + 1 <= CPW - 1)
        def _():
            pltpu.sync_copy(sd_hbm.at[c0 + c + 1], sdv[s1])
            gather(c + 1, s1).start()

        et_copy(c).wait()
        gather(c, s).wait()

        def row(r, _):
            for jj in range(H // 16):
                sl = pl.ds(jj * 16, 16)
                etb[r, sl] = jnp.maximum(zb[s][r, sl] + etb[r, sl], 0.0)
            return 0

        lax.fori_loop(0, KE, row, 0)

        @pl.when(gc < NREAL)
        def _():
            pltpu.sync_copy(etb, agg_sh.at[sdv[s].at[1]], add=True)

        @pl.when(c + 1 <= CPW - 1)
        def _():
            et_copy(c + 1).start()

    def pair(j, _):
        phase(2 * j, 0)
        phase(2 * j + 1, 1)
        return 0

    lax.fori_loop(0, CPW // 2, pair, 0)
    plsc.subcore_barrier()

    for k in range(4):
        pltpu.sync_copy(agg_sh.at[pl.ds(r0 + k * KE, KE)],
                        out_hbm.at[cid, pl.ds(r0 + k * KE, KE)])
    tail = ROWS_PER_SUB - 4 * KE
    pltpu.sync_copy(agg_sh.at[pl.ds(r0 + 4 * KE, tail)],
                    out_hbm.at[cid, pl.ds(r0 + 4 * KE, tail)])

    @pl.when(sid == 15)
    def _():
        lax.fori_loop(0, KE, zrow, 0)
        pltpu.sync_copy(etb, out_hbm.at[cid, pl.ds(NPA, NP - NPA)])


def _sc_msg(z, et, sd):
    k = pl.kernel(
        _sc_msg_body,
        out_type=jax.ShapeDtypeStruct((2, NP, H), jnp.float32),
        mesh=plsc.VectorSubcoreMesh(core_axis_name="c", subcore_axis_name="s"),
        scratch_types=[
            [pltpu.VMEM((2, KE), jnp.int32) for _ in range(2)],
            pltpu.VMEM((KE, H), jnp.float32),
            [pltpu.VMEM((KE, H), jnp.float32) for _ in range(2)],
            pltpu.VMEM_SHARED((NPA, H), jnp.float32),
            [pltpu.SemaphoreType.DMA for _ in range(2)],
            pltpu.SemaphoreType.DMA,
        ],
    )
    return k(z, et, sd)


def _update_core(x_ref, a_ref, bat_ref, wux_ref, wua_ref, bu_ref):
    x = x_ref[...]
    a = a_ref[0] + a_ref[1]
    xn = jnp.dot(x, wux_ref[...], preferred_element_type=jnp.float32)
    xn = xn + jnp.dot(a, wua_ref[...], preferred_element_type=jnp.float32)
    xn = jnp.maximum(xn + bu_ref[...], 0.0)
    ids = bat_ref[...]
    oh = (ids == lax.broadcasted_iota(jnp.int32, (TB, G), 1)).astype(jnp.float32)
    contrib = lax.dot_general(oh, xn, (((0,), (0,)), ((), ())),
                              preferred_element_type=jnp.float32)
    return xn, contrib


def _upd_body(x_ref, a_ref, bat_ref, wux_ref, wua_ref, bu_ref,
              wmx_ref, xn_ref, z2_ref, pool_ref):
    xn, contrib = _update_core(x_ref, a_ref, bat_ref, wux_ref, wua_ref, bu_ref)
    xn_ref[...] = xn
    z2_ref[...] = jnp.dot(xn, wmx_ref[...], preferred_element_type=jnp.float32)

    @pl.when(pl.program_id(0) == 0)
    def _():
        pool_ref[...] = jnp.zeros_like(pool_ref)

    pool_ref[...] += contrib


def _upd_pool_body(x_ref, a_ref, bat_ref, wux_ref, wua_ref, bu_ref, pool_ref):
    _, contrib = _update_core(x_ref, a_ref, bat_ref, wux_ref, wua_ref, bu_ref)

    @pl.when(pl.program_id(0) == 0)
    def _():
        pool_ref[...] = jnp.zeros_like(pool_ref)

    pool_ref[...] += contrib


def _update(x_p, agg2, bat_p, wux, wua, bu, wmx_next):
    grid = NP // TB
    nspec = pl.BlockSpec((TB, H), lambda i: (i, 0))
    aspec = pl.BlockSpec((2, TB, H), lambda i: (0, i, 0))
    wspec = pl.BlockSpec((H, H), lambda i: (0, 0))
    pspec = pl.BlockSpec((G, H), lambda i: (0, 0))
    in_specs = [
        nspec, aspec,
        pl.BlockSpec((TB, 1), lambda i: (i, 0)),
        wspec, wspec,
        pl.BlockSpec((1, H), lambda i: (0, 0)),
    ]
    if wmx_next is not None:
        return pl.pallas_call(
            _upd_body,
            grid=(grid,),
            in_specs=in_specs + [wspec],
            out_specs=[nspec, nspec, pspec],
            out_shape=[
                jax.ShapeDtypeStruct((NP, H), jnp.float32),
                jax.ShapeDtypeStruct((NP, H), jnp.float32),
                jax.ShapeDtypeStruct((G, H), jnp.float32),
            ],
        )(x_p, agg2, bat_p, wux, wua, bu, wmx_next)
    return pl.pallas_call(
        _upd_pool_body,
        grid=(grid,),
        in_specs=in_specs,
        out_specs=pspec,
        out_shape=jax.ShapeDtypeStruct((G, H), jnp.float32),
    )(x_p, agg2, bat_p, wux, wua, bu)


def _head_body(p0_ref, p1_ref, p2_ref, bat_ref, w1_ref, b1_ref,
               wf_ref, bf_ref, out_ref):
    ids = bat_ref[...]
    oh = (ids == lax.broadcasted_iota(jnp.int32, (NP, G), 1)).astype(jnp.float32)
    counts = jnp.maximum(jnp.sum(oh, axis=0), 1.0)
    inv = (1.0 / counts)[:, None]
    h = jnp.concatenate([p0_ref[...] * inv, p1_ref[...] * inv, p2_ref[...] * inv],
                        axis=1)
    y = jnp.dot(h, w1_ref[...], preferred_element_type=jnp.float32) + b1_ref[...]
    y = jnp.where(y >= 0.0, y, SLOPE * y)
    out_ref[...] = jnp.dot(y, wf_ref[...], preferred_element_type=jnp.float32) + bf_ref[...]


def _head(p0, p1, p2, bat_p, w1, b1, wf_p, bf_p):
    return pl.pallas_call(
        _head_body,
        out_shape=jax.ShapeDtypeStruct((G, 128), jnp.float32),
    )(p0, p1, p2, bat_p, w1, b1, wf_p, bf_p)


def kernel(x, edge_index, edge_attr, batch,
           ln_node_g, ln_node_b, ln_edge_g, ln_edge_b,
           W_msg0, b_msg0, W_upd0, b_upd0,
           W_msg1, b_msg1, W_upd1, b_upd1,
           W_msg2, b_msg2, W_upd2, b_upd2,
           W_lin1, b_lin1, W_fin, b_fin):
    f32 = jnp.float32
    x_p = jnp.pad(x, ((0, NP - N), (0, 0)))
    ea_p = jnp.pad(edge_attr, ((0, EP - E), (0, 0)))
    pad_idx = (jnp.arange(EP - E, dtype=jnp.int32) * 97) % N
    src_p = jnp.concatenate([edge_index[0], pad_idx])
    dst_p = jnp.concatenate([edge_index[1], pad_idx])
    sd = jnp.stack([src_p.reshape(-1, KE), dst_p.reshape(-1, KE)], axis=1)
    bat_p = jnp.pad(batch, (0, NP - N), constant_values=G)[:, None]

    g_n = ln_node_g[None, :].astype(f32)
    b_n = ln_node_b[None, :].astype(f32)
    g_e = ln_edge_g[None, :].astype(f32)
    b_e = ln_edge_b[None, :].astype(f32)

    wmx = [W_msg0[:D], W_msg1[:D], W_msg2[:D]]
    wme = [W_msg0[D:], W_msg1[D:], W_msg2[D:]]
    bm = [b_msg0[None, :], b_msg1[None, :], b_msg2[None, :]]
    wux = [W_upd0[:D], W_upd1[:D], W_upd2[:D]]
    wua = [W_upd0[D:], W_upd1[D:], W_upd2[D:]]
    bu = [b_upd0[None, :], b_upd1[None, :], b_upd2[None, :]]

    x0, z = _ln_z(x_p, g_n, b_n, wmx[0])

    pooled = []
    x_cur = x0
    for l in range(3):
        et = _eterm(ea_p, g_e, b_e, wme[l], bm[l])
        aggp = _sc_msg(z, et, sd)
        if l < 2:
            x_cur, z, pool = _update(x_cur, aggp, bat_p,
                                     wux[l], wua[l], bu[l], wmx[l + 1])
        else:
            pool = _update(x_cur, aggp, bat_p, wux[l], wua[l], bu[l], None)
        pooled.append(pool)

    wf_p = jnp.pad(W_fin, ((0, 0), (0, 128 - C)))
    bf_p = jnp.pad(b_fin, (0, 128 - C))[None, :]
    out = _head(pooled[0], pooled[1], pooled[2], bat_p,
                W_lin1, b_lin1[None, :], wf_p, bf_p)
    return out[:, :C]


# speedup vs baseline: 3.8747x; 1.0439x over previous
import jax
import jax.numpy as jnp
from jax import lax
from jax.experimental import pallas as pl
from jax.experimental.pallas import tpu as pltpu
from jax.experimental.pallas import tpu_sc as plsc

N = 10000
NP = 10240
E = 320000
EP = 327680
D = 128
DE = 16
H = 128
HH = H // 2
G = 64
C = 10
SLOPE = 0.2

KE = 128
CPW = 80
NREAL = E // KE
NPA = 10112
ROWS_PER_SUB = NPA // 16
TB = 512
TEB = 2560


def _ln_z_body(x_ref, g_ref, b_ref, wmx_ref, x0_ref, z_ref):
    x = x_ref[...]
    mu = jnp.mean(x, axis=1, keepdims=True)
    var = jnp.mean((x - mu) ** 2, axis=1, keepdims=True)
    xn = (x - mu) * jax.lax.rsqrt(var + 1e-5) * g_ref[...] + b_ref[...]
    x0_ref[...] = xn
    z_ref[...] = jnp.dot(xn, wmx_ref[...], preferred_element_type=jnp.float32)


def _ln_z(x_p, g, b, wmx):
    grid = NP // TB
    return pl.pallas_call(
        _ln_z_body,
        grid=(grid,),
        in_specs=[
            pl.BlockSpec((TB, D), lambda i: (i, 0)),
            pl.BlockSpec((1, D), lambda i: (0, 0)),
            pl.BlockSpec((1, D), lambda i: (0, 0)),
            pl.BlockSpec((D, H), lambda i: (0, 0)),
        ],
        out_specs=[
            pl.BlockSpec((TB, D), lambda i: (i, 0)),
            pl.BlockSpec((TB, H), lambda i: (i, 0)),
        ],
        out_shape=[
            jax.ShapeDtypeStruct((NP, D), jnp.float32),
            jax.ShapeDtypeStruct((NP, H), jnp.float32),
        ],
    )(x_p, g, b, wmx)


def _eterm_body(ea_ref, g_ref, b_ref, w_ref, bm_ref, e2_ref):
    ea = ea_ref[...]
    mu = jnp.mean(ea, axis=1, keepdims=True)
    var = jnp.mean((ea - mu) ** 2, axis=1, keepdims=True)
    ean = (ea - mu) * jax.lax.rsqrt(var + 1e-5) * g_ref[...] + b_ref[...]
    e2_ref[...] = jnp.dot(ean, w_ref[...], preferred_element_type=jnp.float32) + bm_ref[...]


def _eterm(ea, g, b, we, bm):
    grid = E // TEB
    return pl.pallas_call(
        _eterm_body,
        grid=(grid,),
        in_specs=[
            pl.BlockSpec((TEB, DE), lambda i: (i, 0)),
            pl.BlockSpec((1, DE), lambda i: (0, 0)),
            pl.BlockSpec((1, DE), lambda i: (0, 0)),
            pl.BlockSpec((DE, H), lambda i: (0, 0)),
            pl.BlockSpec((1, H), lambda i: (0, 0)),
        ],
        out_specs=pl.BlockSpec((TEB, H), lambda i: (i, 0)),
        out_shape=jax.ShapeDtypeStruct((EP, H), jnp.float32),
    )(ea, g, b, we, bm)


def _sc_msg_body(z_hbm, et_hbm, sd_hbm, out_hbm,
                 sdv, etb, zb, agg_sh, sem_g, sem_et):
    cid = lax.axis_index("c")
    sid = lax.axis_index("s")
    w = cid * 16 + sid
    c0 = w * CPW
    zv = jnp.zeros((16,), jnp.float32)

    def zrow(r, _):
        for j in range(H // 16):
            etb[r, pl.ds(j * 16, 16)] = zv
        return 0

    lax.fori_loop(0, KE, zrow, 0)
    r0 = sid * ROWS_PER_SUB
    for k in range(4):
        pltpu.sync_copy(etb, agg_sh.at[pl.ds(r0 + k * KE, KE)])
    pltpu.sync_copy(etb.at[pl.ds(0, ROWS_PER_SUB - 4 * KE)],
                    agg_sh.at[pl.ds(r0 + 4 * KE, ROWS_PER_SUB - 4 * KE)])
    plsc.subcore_barrier()

    def et_copy(c):
        return pltpu.make_async_copy(
            et_hbm.at[pl.ds((c0 + c) * KE, KE)], etb, sem_et)

    def gather(c, s):
        return pltpu.make_async_copy(z_hbm.at[sdv[s].at[0]], zb[s], sem_g[s])

    pltpu.sync_copy(sd_hbm.at[c0], sdv[0])
    gather(0, 0).start()
    et_copy(0).start()

    def phase(c, s):
        gc = c0 + c
        s1 = 1 - s

        @pl.when(c + 1 <= CPW - 1)
        def _():
            pltpu.sync_copy(sd_hbm.at[c0 + c + 1], sdv[s1])
            gather(c + 1, s1).start()

        et_copy(c).wait()
        gather(c, s).wait()

        def row(r, _):
            for jj in range(H // 16):
                sl = pl.ds(jj * 16, 16)
                etb[r, sl] = jnp.maximum(zb[s][r, sl] + etb[r, sl], 0.0)
            return 0

        lax.fori_loop(0, KE, row, 0)

        @pl.when(gc < NREAL)
        def _():
            pltpu.sync_copy(etb, agg_sh.at[sdv[s].at[1]], add=True)

        @pl.when(c + 1 <= CPW - 1)
        def _():
            et_copy(c + 1).start()

    def pair(j, _):
        phase(2 * j, 0)
        phase(2 * j + 1, 1)
        return 0

    lax.fori_loop(0, CPW // 2, pair, 0)
    plsc.subcore_barrier()

    for k in range(4):
        pltpu.sync_copy(agg_sh.at[pl.ds(r0 + k * KE, KE)],
                        out_hbm.at[cid, pl.ds(r0 + k * KE, KE)])
    tail = ROWS_PER_SUB - 4 * KE
    pltpu.sync_copy(agg_sh.at[pl.ds(r0 + 4 * KE, tail)],
                    out_hbm.at[cid, pl.ds(r0 + 4 * KE, tail)])

    @pl.when(sid == 15)
    def _():
        lax.fori_loop(0, KE, zrow, 0)
        pltpu.sync_copy(etb, out_hbm.at[cid, pl.ds(NPA, NP - NPA)])


def _sc_msg(z, et, sd):
    k = pl.kernel(
        _sc_msg_body,
        out_type=jax.ShapeDtypeStruct((2, NP, H), jnp.float32),
        mesh=plsc.VectorSubcoreMesh(core_axis_name="c", subcore_axis_name="s"),
        scratch_types=[
            [pltpu.VMEM((2, KE), jnp.int32) for _ in range(2)],
            pltpu.VMEM((KE, H), jnp.float32),
            [pltpu.VMEM((KE, H), jnp.float32) for _ in range(2)],
            pltpu.VMEM_SHARED((NPA, H), jnp.float32),
            [pltpu.SemaphoreType.DMA for _ in range(2)],
            pltpu.SemaphoreType.DMA,
        ],
    )
    return k(z, et, sd)


def _update_core(x_ref, a_ref, bat_ref, wux_ref, wua_ref, bu_ref):
    x = x_ref[...]
    a = a_ref[0] + a_ref[1]
    xn = jnp.dot(x, wux_ref[...], preferred_element_type=jnp.float32)
    xn = xn + jnp.dot(a, wua_ref[...], preferred_element_type=jnp.float32)
    xn = jnp.maximum(xn + bu_ref[...], 0.0)
    ids = bat_ref[...]
    oh = (ids == lax.broadcasted_iota(jnp.int32, (TB, G), 1)).astype(jnp.float32)
    contrib = lax.dot_general(oh, xn, (((0,), (0,)), ((), ())),
                              preferred_element_type=jnp.float32)
    return xn, contrib


def _upd_body(x_ref, a_ref, bat_ref, wux_ref, wua_ref, bu_ref,
              wmx_ref, xn_ref, z2_ref, pool_ref):
    xn, contrib = _update_core(x_ref, a_ref, bat_ref, wux_ref, wua_ref, bu_ref)
    xn_ref[...] = xn
    z2_ref[...] = jnp.dot(xn, wmx_ref[...], preferred_element_type=jnp.float32)

    @pl.when(pl.program_id(0) == 0)
    def _():
        pool_ref[...] = jnp.zeros_like(pool_ref)

    pool_ref[...] += contrib


def _upd_pool_body(x_ref, a_ref, bat_ref, wux_ref, wua_ref, bu_ref, pool_ref):
    _, contrib = _update_core(x_ref, a_ref, bat_ref, wux_ref, wua_ref, bu_ref)

    @pl.when(pl.program_id(0) == 0)
    def _():
        pool_ref[...] = jnp.zeros_like(pool_ref)

    pool_ref[...] += contrib


def _update(x_p, agg2, bat_p, wux, wua, bu, wmx_next):
    grid = NP // TB
    nspec = pl.BlockSpec((TB, H), lambda i: (i, 0))
    aspec = pl.BlockSpec((2, TB, H), lambda i: (0, i, 0))
    wspec = pl.BlockSpec((H, H), lambda i: (0, 0))
    pspec = pl.BlockSpec((G, H), lambda i: (0, 0))
    in_specs = [
        nspec, aspec,
        pl.BlockSpec((TB, 1), lambda i: (i, 0)),
        wspec, wspec,
        pl.BlockSpec((1, H), lambda i: (0, 0)),
    ]
    if wmx_next is not None:
        return pl.pallas_call(
            _upd_body,
            grid=(grid,),
            in_specs=in_specs + [wspec],
            out_specs=[nspec, nspec, pspec],
            out_shape=[
                jax.ShapeDtypeStruct((NP, H), jnp.float32),
                jax.ShapeDtypeStruct((NP, H), jnp.float32),
                jax.ShapeDtypeStruct((G, H), jnp.float32),
            ],
        )(x_p, agg2, bat_p, wux, wua, bu, wmx_next)
    return pl.pallas_call(
        _upd_pool_body,
        grid=(grid,),
        in_specs=in_specs,
        out_specs=pspec,
        out_shape=jax.ShapeDtypeStruct((G, H), jnp.float32),
    )(x_p, agg2, bat_p, wux, wua, bu)


def _head_body(p0_ref, p1_ref, p2_ref, bat_ref, w1_ref, b1_ref,
               wf_ref, bf_ref, out_ref):
    ids = bat_ref[...]
    oh = (ids == lax.broadcasted_iota(jnp.int32, (NP, G), 1)).astype(jnp.float32)
    counts = jnp.maximum(jnp.sum(oh, axis=0), 1.0)
    inv = (1.0 / counts)[:, None]
    h = jnp.concatenate([p0_ref[...] * inv, p1_ref[...] * inv, p2_ref[...] * inv],
                        axis=1)
    y = jnp.dot(h, w1_ref[...], preferred_element_type=jnp.float32) + b1_ref[...]
    y = jnp.where(y >= 0.0, y, SLOPE * y)
    out_ref[...] = jnp.dot(y, wf_ref[...], preferred_element_type=jnp.float32) + bf_ref[...]


def _head(p0, p1, p2, bat_p, w1, b1, wf_p, bf_p):
    return pl.pallas_call(
        _head_body,
        out_shape=jax.ShapeDtypeStruct((G, 128), jnp.float32),
    )(p0, p1, p2, bat_p, w1, b1, wf_p, bf_p)


def kernel(x, edge_index, edge_attr, batch,
           ln_node_g, ln_node_b, ln_edge_g, ln_edge_b,
           W_msg0, b_msg0, W_upd0, b_upd0,
           W_msg1, b_msg1, W_upd1, b_upd1,
           W_msg2, b_msg2, W_upd2, b_upd2,
           W_lin1, b_lin1, W_fin, b_fin):
    f32 = jnp.float32
    x_p = jnp.pad(x, ((0, NP - N), (0, 0)))
    pad_idx = (jnp.arange(EP - E, dtype=jnp.int32) * 97) % N
    src_p = jnp.concatenate([edge_index[0], pad_idx])
    dst_p = jnp.concatenate([edge_index[1], pad_idx])
    sd = jnp.stack([src_p.reshape(-1, KE), dst_p.reshape(-1, KE)], axis=1)
    bat_p = jnp.pad(batch, (0, NP - N), constant_values=G)[:, None]

    g_n = ln_node_g[None, :].astype(f32)
    b_n = ln_node_b[None, :].astype(f32)
    g_e = ln_edge_g[None, :].astype(f32)
    b_e = ln_edge_b[None, :].astype(f32)

    wmx = [W_msg0[:D], W_msg1[:D], W_msg2[:D]]
    wme = [W_msg0[D:], W_msg1[D:], W_msg2[D:]]
    bm = [b_msg0[None, :], b_msg1[None, :], b_msg2[None, :]]
    wux = [W_upd0[:D], W_upd1[:D], W_upd2[:D]]
    wua = [W_upd0[D:], W_upd1[D:], W_upd2[D:]]
    bu = [b_upd0[None, :], b_upd1[None, :], b_upd2[None, :]]

    x0, z = _ln_z(x_p, g_n, b_n, wmx[0])
    ets = [_eterm(edge_attr, g_e, b_e, wme[l], bm[l]) for l in range(3)]

    pooled = []
    x_cur = x0
    for l in range(3):
        aggp = _sc_msg(z, ets[l], sd)
        if l < 2:
            x_cur, z, pool = _update(x_cur, aggp, bat_p,
                                     wux[l], wua[l], bu[l], wmx[l + 1])
        else:
            pool = _update(x_cur, aggp, bat_p, wux[l], wua[l], bu[l], None)
        pooled.append(pool)

    wf_p = jnp.pad(W_fin, ((0, 0), (0, 128 - C)))
    bf_p = jnp.pad(b_fin, (0, 128 - C))[None, :]
    out = _head(pooled[0], pooled[1], pooled[2], bat_p,
                W_lin1, b_lin1[None, :], wf_p, bf_p)
    return out[:, :C]
